# baseline (speedup 1.0000x reference)
import functools

import jax
import jax.numpy as jnp
from jax import lax
from jax.experimental import pallas as pl
from jax.experimental.pallas import tpu as pltpu
from jax.experimental.pallas import tpu_sc as plsc

B = 16384
DSET = 1000
DOM = 100
M = 16
D = 128

NC = 2
NS = 16
NW = NC * NS
BPW = B // NW
L = 16

C1 = 16
C2 = 128
N1 = BPW // C1
N2 = BPW // C2


def _sc_body(ds_hbm, dom_hbm, sys_hbm, domtab_hbm, out1_hbm, out2_hbm,
             ds_v, dom_v, flat_v, buf1, buf2, sem_in, sem_out):
    wid = lax.axis_index("s") * NC + lax.axis_index("c")
    base = wid * BPW

    pltpu.sync_copy(ds_hbm.at[pl.ds(base, BPW)], ds_v)
    pltpu.sync_copy(dom_hbm.at[pl.ds(base, BPW)], dom_v)

    for i in range(BPW // L):
        sl = pl.ds(i * L, L)
        flat_v[sl] = ds_v[sl] * DOM + dom_v[sl]

    g = pltpu.async_copy(domtab_hbm.at[flat_v.at[pl.ds(0, C2)]],
                         buf2.at[0], sem_in)
    writes = []
    for c in range(N2):
        g.wait()
        if c + 1 < N2:
            g = pltpu.async_copy(
                domtab_hbm.at[flat_v.at[pl.ds((c + 1) * C2, C2)]],
                buf2.at[(c + 1) % 2], sem_in)
        if len(writes) == 2:
            writes.pop(0).wait()
        writes.append(pltpu.async_copy(
            buf2.at[c % 2], out2_hbm.at[pl.ds(base + c * C2, C2)], sem_out))
    for w in writes:
        w.wait()

    g = pltpu.async_copy(sys_hbm.at[ds_v.at[pl.ds(0, C1)]],
                         buf1.at[0], sem_in)
    writes = []
    for c in range(N1):
        g.wait()
        if c + 1 < N1:
            g = pltpu.async_copy(
                sys_hbm.at[ds_v.at[pl.ds((c + 1) * C1, C1)]],
                buf1.at[(c + 1) % 2], sem_in)
        if len(writes) == 2:
            writes.pop(0).wait()
        writes.append(pltpu.async_copy(
            buf1.at[c % 2], out1_hbm.at[pl.ds(base + c * C1, C1)], sem_out))
    for w in writes:
        w.wait()


@jax.jit
def _sc_call(dataset_id, domain_id, sys_flat, dom_flat):
    mesh = plsc.VectorSubcoreMesh(core_axis_name="c", subcore_axis_name="s",
                                  num_cores=NC, num_subcores=NS)
    return pl.kernel(
        _sc_body,
        out_type=(
            jax.ShapeDtypeStruct((B, M * D), jnp.float32),
            jax.ShapeDtypeStruct((B, D), jnp.float32),
        ),
        mesh=mesh,
        scratch_types=[
            pltpu.VMEM((BPW,), jnp.int32),
            pltpu.VMEM((BPW,), jnp.int32),
            pltpu.VMEM((BPW,), jnp.int32),
            pltpu.VMEM((2, C1, M * D), jnp.float32),
            pltpu.VMEM((2, C2, D), jnp.float32),
            pltpu.SemaphoreType.DMA,
            pltpu.SemaphoreType.DMA,
        ],
    )(dataset_id, domain_id, sys_flat, dom_flat)


def kernel(Dataset_id, Domain_id, system_prompts, domain_prompts,
           phys_dataset_emb, phys_domain_emb):
    del phys_dataset_emb, phys_domain_emb
    sys_flat = system_prompts.reshape(DSET, M * D)
    dom_flat = domain_prompts.reshape(DSET * DOM, D)
    out1, out2 = _sc_call(Dataset_id, Domain_id, sys_flat, dom_flat)
    return out1.reshape(B, M, D), out2
